# baseline (device time: 194447 ns/iter reference)
import functools

import jax
import jax.numpy as jnp
from jax import lax
from jax.experimental import pallas as pl
from jax.experimental.pallas import tpu as pltpu

N_DEV = 8
M = 2048
D = 2048
CHUNK = M // N_DEV
HALF = CHUNK // 2
Q = 4
SUB = HALF // Q
N_LANE = 2 * Q
N_HOP = N_DEV - 1


def kernel(partial, resid, gamma):
    partial2d = partial.reshape(M, D)
    gamma2d = gamma.reshape(1, D)

    def body(
        partial_ref,
        resid_ref,
        gamma_ref,
        out_ref,
        comm_rs,
        comm_ag,
        stage_send,
        stage_acc,
        resid_stage,
        rs_send_sems,
        rs_recv_sems,
        ag_send_sems,
        ag_recv_sems,
        cp_send_sem,
        cp_acc_sems,
        cp_resid_sems,
    ):
        my = lax.axis_index("i")
        left = lax.rem(my - 1 + N_DEV, N_DEV)
        right = lax.rem(my + 1, N_DEV)
        lane_dir = [l // Q for l in range(N_LANE)]
        peer = [(right, left)[d] for d in lane_dir]
        own = [(right, left)[d] for d in lane_dir]

        def mod(c):
            return lax.rem(c + 2 * N_DEV, N_DEV)

        def lrows(c, l):
            off = lane_dir[l] * HALF + (l % Q) * SUB
            return pl.ds(c * CHUNK + off, SUB)

        def recv_chunk(l, s):
            return mod(my - (1 - 2 * lane_dir[l]) * (s + 1))

        def stage_cp(l, s):
            return pltpu.make_async_copy(
                partial_ref.at[lrows(recv_chunk(l, s), l), :],
                stage_acc.at[l, s % 2],
                cp_acc_sems.at[l, s % 2],
            )

        cp_send = pltpu.make_async_copy(
            partial_ref.at[pl.ds(my * CHUNK, CHUNK), :],
            stage_send,
            cp_send_sem,
        )
        cp_send.start()
        cp_resid = []
        for l in range(N_LANE):
            cp = pltpu.make_async_copy(
                resid_ref.at[lrows(own[l], l), :],
                resid_stage.at[l],
                cp_resid_sems.at[l],
            )
            cp.start()
            cp_resid.append(cp)
        for l in range(N_LANE):
            stage_cp(l, 0).start()

        barrier_sem = pltpu.get_barrier_semaphore()
        for nbr in (left, right):
            pl.semaphore_signal(
                barrier_sem,
                inc=1,
                device_id=(nbr,),
                device_id_type=pl.DeviceIdType.MESH,
            )
        pl.semaphore_wait(barrier_sem, 2)

        cp_send.wait()

        def rs_rdma(l, s, src):
            return pltpu.make_async_remote_copy(
                src_ref=src,
                dst_ref=comm_rs.at[l, s],
                send_sem=rs_send_sems.at[l, s],
                recv_sem=rs_recv_sems.at[l, s],
                device_id=(peer[l],),
                device_id_type=pl.DeviceIdType.MESH,
            )

        rs = [[None] * N_HOP for _ in range(N_LANE)]
        for l in range(N_LANE):
            off = lane_dir[l] * HALF + (l % Q) * SUB
            rs[l][0] = rs_rdma(l, 0, stage_send.at[pl.ds(off, SUB), :])
            rs[l][0].start()
        for s in range(N_HOP):
            if s + 1 < N_HOP:
                for l in range(N_LANE):
                    stage_cp(l, s + 1).start()
            for l in range(N_LANE):
                rs[l][s].wait_recv()
                stage_cp(l, s).wait()
                rc = recv_chunk(l, s)
                out_ref[lrows(rc, l), :] = (
                    comm_rs[l, s] + stage_acc[l, s % 2, :, :]
                )
                if s + 1 < N_HOP:
                    rs[l][s + 1] = rs_rdma(l, s + 1, out_ref.at[lrows(rc, l), :])
                    rs[l][s + 1].start()
        for l in range(N_LANE):
            for s in range(N_HOP):
                rs[l][s].wait_send()

        def ag_rdma(l, t, src):
            return pltpu.make_async_remote_copy(
                src_ref=src,
                dst_ref=comm_ag.at[l, t],
                send_sem=ag_send_sems.at[l, t],
                recv_sem=ag_recv_sems.at[l, t],
                device_id=(peer[l],),
                device_id_type=pl.DeviceIdType.MESH,
            )

        ag = [[None] * N_HOP for _ in range(N_LANE)]
        for l in range(N_LANE):
            cp_resid[l].wait()
            rows = lrows(own[l], l)
            y = out_ref[rows, :] + resid_stage[l, :, :]
            rms = jnp.sqrt(jnp.mean(y * y, axis=-1, keepdims=True) + 1e-6)
            out_ref[rows, :] = y / rms * gamma_ref[:, :]
            ag[l][0] = ag_rdma(l, 0, out_ref.at[rows, :])
            ag[l][0].start()

        for t in range(N_HOP):
            for l in range(N_LANE):
                ag[l][t].wait_recv()
                if t + 1 < N_HOP:
                    ag[l][t + 1] = ag_rdma(l, t + 1, comm_ag.at[l, t])
                    ag[l][t + 1].start()
                rc = mod(my - (1 - 2 * lane_dir[l]) * t)
                out_ref[lrows(rc, l), :] = comm_ag[l, t]
        for l in range(N_LANE):
            for t in range(N_HOP):
                ag[l][t].wait_send()

        @functools.partial(
            pl.run_scoped, sem=pltpu.SemaphoreType.REGULAR
        )
        def _(sem):
            for nbr in (left, right):
                pl.semaphore_signal(
                    sem,
                    inc=1,
                    device_id=(nbr,),
                    device_id_type=pl.DeviceIdType.MESH,
                )
            pl.semaphore_wait(sem, 2)

    return pl.pallas_call(
        body,
        out_shape=jax.ShapeDtypeStruct((M, D), jnp.float32),
        in_specs=[
            pl.BlockSpec(memory_space=pl.ANY),
            pl.BlockSpec(memory_space=pl.ANY),
            pl.BlockSpec(memory_space=pltpu.VMEM),
        ],
        out_specs=pl.BlockSpec(memory_space=pltpu.VMEM),
        scratch_shapes=[
            pltpu.VMEM((N_LANE, N_HOP, SUB, D), jnp.float32),
            pltpu.VMEM((N_LANE, N_HOP, SUB, D), jnp.float32),
            pltpu.VMEM((CHUNK, D), jnp.float32),
            pltpu.VMEM((N_LANE, 2, SUB, D), jnp.float32),
            pltpu.VMEM((N_LANE, SUB, D), jnp.float32),
            pltpu.SemaphoreType.DMA((N_LANE, N_HOP)),
            pltpu.SemaphoreType.DMA((N_LANE, N_HOP)),
            pltpu.SemaphoreType.DMA((N_LANE, N_HOP)),
            pltpu.SemaphoreType.DMA((N_LANE, N_HOP)),
            pltpu.SemaphoreType.DMA,
            pltpu.SemaphoreType.DMA((N_LANE, 2)),
            pltpu.SemaphoreType.DMA((N_LANE,)),
        ],
        compiler_params=pltpu.CompilerParams(
            collective_id=0,
            vmem_limit_bytes=60 * 1024 * 1024,
        ),
    )(partial2d, resid, gamma2d)


# device time: 190054 ns/iter; 1.0231x vs baseline; 1.0231x over previous
import functools

import jax
import jax.numpy as jnp
from jax import lax
from jax.experimental import pallas as pl
from jax.experimental.pallas import tpu as pltpu

N_DEV = 8
M = 2048
D = 2048
CHUNK = M // N_DEV
HALF = CHUNK // 2
Q = 4
SUB = HALF // Q
N_LANE = 2 * Q
N_HOP = N_DEV - 1


def kernel(partial, resid, gamma):
    gamma2d = gamma.reshape(1, D)

    def body(
        partial_ref,
        resid_ref,
        gamma_ref,
        out_ref,
        comm_rs,
        comm_ag,
        stage_send,
        stage_acc,
        resid_stage,
        rs_send_sems,
        rs_recv_sems,
        ag_send_sems,
        ag_recv_sems,
        cp_send_sem,
        cp_acc_sems,
        cp_resid_sems,
    ):
        my = lax.axis_index("i")
        left = lax.rem(my - 1 + N_DEV, N_DEV)
        right = lax.rem(my + 1, N_DEV)
        lane_dir = [l // Q for l in range(N_LANE)]
        peer = [(right, left)[d] for d in lane_dir]
        own = [(right, left)[d] for d in lane_dir]

        def mod(c):
            return lax.rem(c + 2 * N_DEV, N_DEV)

        def lrows(c, l):
            off = lane_dir[l] * HALF + (l % Q) * SUB
            return pl.ds(c * CHUNK + off, SUB)

        def recv_chunk(l, s):
            return mod(my - (1 - 2 * lane_dir[l]) * (s + 1))

        def stage_cp(l, s):
            return pltpu.make_async_copy(
                partial_ref.at[0, lrows(recv_chunk(l, s), l), :],
                stage_acc.at[l, s % 2],
                cp_acc_sems.at[l, s % 2],
            )

        cp_send = pltpu.make_async_copy(
            partial_ref.at[0, pl.ds(my * CHUNK, CHUNK), :],
            stage_send,
            cp_send_sem,
        )
        cp_send.start()
        cp_resid = []
        for l in range(N_LANE):
            cp = pltpu.make_async_copy(
                resid_ref.at[lrows(own[l], l), :],
                resid_stage.at[l],
                cp_resid_sems.at[l],
            )
            cp.start()
            cp_resid.append(cp)
        for l in range(N_LANE):
            stage_cp(l, 0).start()

        barrier_sem = pltpu.get_barrier_semaphore()
        for nbr in (left, right):
            pl.semaphore_signal(
                barrier_sem,
                inc=1,
                device_id=(nbr,),
                device_id_type=pl.DeviceIdType.MESH,
            )
        pl.semaphore_wait(barrier_sem, 2)

        cp_send.wait()

        def rs_rdma(l, s, src):
            return pltpu.make_async_remote_copy(
                src_ref=src,
                dst_ref=comm_rs.at[l, s],
                send_sem=rs_send_sems.at[l, s],
                recv_sem=rs_recv_sems.at[l, s],
                device_id=(peer[l],),
                device_id_type=pl.DeviceIdType.MESH,
            )

        def ag_rdma(l, t, src):
            return pltpu.make_async_remote_copy(
                src_ref=src,
                dst_ref=comm_ag.at[l, t],
                send_sem=ag_send_sems.at[l, t],
                recv_sem=ag_recv_sems.at[l, t],
                device_id=(peer[l],),
                device_id_type=pl.DeviceIdType.MESH,
            )

        rs = [[None] * N_HOP for _ in range(N_LANE)]
        ag = [[None] * N_HOP for _ in range(N_LANE)]
        for l in range(N_LANE):
            off = lane_dir[l] * HALF + (l % Q) * SUB
            rs[l][0] = rs_rdma(l, 0, stage_send.at[pl.ds(off, SUB), :])
            rs[l][0].start()
        for s in range(N_HOP):
            if s + 1 < N_HOP:
                for l in range(N_LANE):
                    stage_cp(l, s + 1).start()
            for l in range(N_LANE):
                rs[l][s].wait_recv()
                stage_cp(l, s).wait()
                rc = recv_chunk(l, s)
                if s + 1 < N_HOP:
                    out_ref[lrows(rc, l), :] = (
                        comm_rs[l, s] + stage_acc[l, s % 2, :, :]
                    )
                    rs[l][s + 1] = rs_rdma(l, s + 1, out_ref.at[lrows(rc, l), :])
                    rs[l][s + 1].start()
                else:
                    cp_resid[l].wait()
                    rows = lrows(own[l], l)
                    y = (
                        comm_rs[l, s]
                        + stage_acc[l, s % 2, :, :]
                        + resid_stage[l, :, :]
                    )
                    rms = jnp.sqrt(
                        jnp.mean(y * y, axis=-1, keepdims=True) + 1e-6
                    )
                    out_ref[rows, :] = y / rms * gamma_ref[:, :]
                    ag[l][0] = ag_rdma(l, 0, out_ref.at[rows, :])
                    ag[l][0].start()
        for l in range(N_LANE):
            for s in range(N_HOP):
                rs[l][s].wait_send()

        for t in range(N_HOP):
            for l in range(N_LANE):
                ag[l][t].wait_recv()
                if t + 1 < N_HOP:
                    ag[l][t + 1] = ag_rdma(l, t + 1, comm_ag.at[l, t])
                    ag[l][t + 1].start()
                rc = mod(my - (1 - 2 * lane_dir[l]) * t)
                out_ref[lrows(rc, l), :] = comm_ag[l, t]
        for l in range(N_LANE):
            for t in range(N_HOP):
                ag[l][t].wait_send()

        @functools.partial(
            pl.run_scoped, sem=pltpu.SemaphoreType.REGULAR
        )
        def _(sem):
            for nbr in (left, right):
                pl.semaphore_signal(
                    sem,
                    inc=1,
                    device_id=(nbr,),
                    device_id_type=pl.DeviceIdType.MESH,
                )
            pl.semaphore_wait(sem, 2)

    return pl.pallas_call(
        body,
        out_shape=jax.ShapeDtypeStruct((M, D), jnp.float32),
        in_specs=[
            pl.BlockSpec(memory_space=pl.ANY),
            pl.BlockSpec(memory_space=pl.ANY),
            pl.BlockSpec(memory_space=pltpu.VMEM),
        ],
        out_specs=pl.BlockSpec(memory_space=pltpu.VMEM),
        scratch_shapes=[
            pltpu.VMEM((N_LANE, N_HOP, SUB, D), jnp.float32),
            pltpu.VMEM((N_LANE, N_HOP, SUB, D), jnp.float32),
            pltpu.VMEM((CHUNK, D), jnp.float32),
            pltpu.VMEM((N_LANE, 2, SUB, D), jnp.float32),
            pltpu.VMEM((N_LANE, SUB, D), jnp.float32),
            pltpu.SemaphoreType.DMA((N_LANE, N_HOP)),
            pltpu.SemaphoreType.DMA((N_LANE, N_HOP)),
            pltpu.SemaphoreType.DMA((N_LANE, N_HOP)),
            pltpu.SemaphoreType.DMA((N_LANE, N_HOP)),
            pltpu.SemaphoreType.DMA,
            pltpu.SemaphoreType.DMA((N_LANE, 2)),
            pltpu.SemaphoreType.DMA((N_LANE,)),
        ],
        compiler_params=pltpu.CompilerParams(
            collective_id=0,
            vmem_limit_bytes=60 * 1024 * 1024,
        ),
    )(partial, resid, gamma2d)


# device time: 182274 ns/iter; 1.0668x vs baseline; 1.0427x over previous
import functools

import jax
import jax.numpy as jnp
from jax import lax
from jax.experimental import pallas as pl
from jax.experimental.pallas import tpu as pltpu

N_DEV = 8
M = 2048
D = 2048
CHUNK = M // N_DEV
HALF = CHUNK // 2
Q = 2
SUB = HALF // Q
N_LANE = 2 * Q
N_HOP = N_DEV - 1


def kernel(partial, resid, gamma):
    gamma2d = gamma.reshape(1, D)

    def body(
        partial_ref,
        resid_ref,
        gamma_ref,
        out_ref,
        acc_ref,
        comm_rs,
        comm_ag,
        stage_send,
        stage_acc,
        resid_stage,
        rs_send_sems,
        rs_recv_sems,
        ag_send_sems,
        ag_recv_sems,
        cp_send_sem,
        cp_acc_sems,
        cp_resid_sems,
        out_own_sems,
        out_ag_sems,
    ):
        def mod(c):
            return lax.rem(c + 2 * N_DEV, N_DEV)

        def l2r(m):
            return lax.bitwise_xor(m, 3 * lax.shift_right_logical(m, 2))

        my = l2r(lax.axis_index("i"))
        left = l2r(mod(my - 1))
        right = l2r(mod(my + 1))
        lane_dir = [l // Q for l in range(N_LANE)]
        peer = [(right, left)[d] for d in lane_dir]
        own = [(mod(my + 1), mod(my - 1))[d] for d in lane_dir]

        def lrows(c, l):
            off = lane_dir[l] * HALF + (l % Q) * SUB
            return pl.ds(c * CHUNK + off, SUB)

        def recv_chunk(l, s):
            return mod(my - (1 - 2 * lane_dir[l]) * (s + 1))

        def stage_cp(l, s):
            return pltpu.make_async_copy(
                partial_ref.at[0, lrows(recv_chunk(l, s), l), :],
                stage_acc.at[l, s % 2],
                cp_acc_sems.at[l, s % 2],
            )

        cp_send = pltpu.make_async_copy(
            partial_ref.at[0, pl.ds(my * CHUNK, CHUNK), :],
            stage_send,
            cp_send_sem,
        )
        cp_send.start()
        cp_resid = []
        for l in range(N_LANE):
            cp = pltpu.make_async_copy(
                resid_ref.at[lrows(own[l], l), :],
                resid_stage.at[l],
                cp_resid_sems.at[l],
            )
            cp.start()
            cp_resid.append(cp)
        for l in range(N_LANE):
            stage_cp(l, 0).start()

        barrier_sem = pltpu.get_barrier_semaphore()
        for nbr in (left, right):
            pl.semaphore_signal(
                barrier_sem,
                inc=1,
                device_id=(nbr,),
                device_id_type=pl.DeviceIdType.MESH,
            )
        pl.semaphore_wait(barrier_sem, 2)

        cp_send.wait()

        def rs_rdma(l, s, src):
            return pltpu.make_async_remote_copy(
                src_ref=src,
                dst_ref=comm_rs.at[l, s],
                send_sem=rs_send_sems.at[l, s],
                recv_sem=rs_recv_sems.at[l, s],
                device_id=(peer[l],),
                device_id_type=pl.DeviceIdType.MESH,
            )

        def ag_rdma(l, t, src):
            return pltpu.make_async_remote_copy(
                src_ref=src,
                dst_ref=comm_ag.at[l, t],
                send_sem=ag_send_sems.at[l, t],
                recv_sem=ag_recv_sems.at[l, t],
                device_id=(peer[l],),
                device_id_type=pl.DeviceIdType.MESH,
            )

        rs = [[None] * N_HOP for _ in range(N_LANE)]
        ag = [[None] * N_HOP for _ in range(N_LANE)]
        out_cps = []
        for l in range(N_LANE):
            off = lane_dir[l] * HALF + (l % Q) * SUB
            rs[l][0] = rs_rdma(l, 0, stage_send.at[pl.ds(off, SUB), :])
            rs[l][0].start()
        for s in range(N_HOP):
            if s + 1 < N_HOP:
                for l in range(N_LANE):
                    stage_cp(l, s + 1).start()
            for l in range(N_LANE):
                rs[l][s].wait_recv()
                stage_cp(l, s).wait()
                rc = recv_chunk(l, s)
                if s + 1 < N_HOP:
                    acc_ref[lrows(rc, l), :] = (
                        comm_rs[l, s] + stage_acc[l, s % 2, :, :]
                    )
                    rs[l][s + 1] = rs_rdma(l, s + 1, acc_ref.at[lrows(rc, l), :])
                    rs[l][s + 1].start()
                else:
                    cp_resid[l].wait()
                    rows = lrows(own[l], l)
                    y = (
                        comm_rs[l, s]
                        + stage_acc[l, s % 2, :, :]
                        + resid_stage[l, :, :]
                    )
                    rms = jnp.sqrt(
                        jnp.mean(y * y, axis=-1, keepdims=True) + 1e-6
                    )
                    acc_ref[rows, :] = y / rms * gamma_ref[:, :]
                    ag[l][0] = ag_rdma(l, 0, acc_ref.at[rows, :])
                    ag[l][0].start()
                    cp = pltpu.make_async_copy(
                        acc_ref.at[rows, :],
                        out_ref.at[rows, :],
                        out_own_sems.at[l],
                    )
                    cp.start()
                    out_cps.append(cp)
        for l in range(N_LANE):
            for s in range(N_HOP):
                rs[l][s].wait_send()

        for t in range(N_HOP):
            for l in range(N_LANE):
                ag[l][t].wait_recv()
                if t + 1 < N_HOP:
                    ag[l][t + 1] = ag_rdma(l, t + 1, comm_ag.at[l, t])
                    ag[l][t + 1].start()
                rc = mod(my - (1 - 2 * lane_dir[l]) * t)
                cp = pltpu.make_async_copy(
                    comm_ag.at[l, t],
                    out_ref.at[lrows(rc, l), :],
                    out_ag_sems.at[l, t],
                )
                cp.start()
                out_cps.append(cp)
        for l in range(N_LANE):
            for t in range(N_HOP):
                ag[l][t].wait_send()
        for cp in out_cps:
            cp.wait()

        @functools.partial(
            pl.run_scoped, sem=pltpu.SemaphoreType.REGULAR
        )
        def _(sem):
            for nbr in (left, right):
                pl.semaphore_signal(
                    sem,
                    inc=1,
                    device_id=(nbr,),
                    device_id_type=pl.DeviceIdType.MESH,
                )
            pl.semaphore_wait(sem, 2)

    return pl.pallas_call(
        body,
        out_shape=jax.ShapeDtypeStruct((M, D), jnp.float32),
        in_specs=[
            pl.BlockSpec(memory_space=pl.ANY),
            pl.BlockSpec(memory_space=pl.ANY),
            pl.BlockSpec(memory_space=pltpu.VMEM),
        ],
        out_specs=pl.BlockSpec(memory_space=pl.ANY),
        scratch_shapes=[
            pltpu.VMEM((M, D), jnp.float32),
            pltpu.VMEM((N_LANE, N_HOP, SUB, D), jnp.float32),
            pltpu.VMEM((N_LANE, N_HOP, SUB, D), jnp.float32),
            pltpu.VMEM((CHUNK, D), jnp.float32),
            pltpu.VMEM((N_LANE, 2, SUB, D), jnp.float32),
            pltpu.VMEM((N_LANE, SUB, D), jnp.float32),
            pltpu.SemaphoreType.DMA((N_LANE, N_HOP)),
            pltpu.SemaphoreType.DMA((N_LANE, N_HOP)),
            pltpu.SemaphoreType.DMA((N_LANE, N_HOP)),
            pltpu.SemaphoreType.DMA((N_LANE, N_HOP)),
            pltpu.SemaphoreType.DMA,
            pltpu.SemaphoreType.DMA((N_LANE, 2)),
            pltpu.SemaphoreType.DMA((N_LANE,)),
            pltpu.SemaphoreType.DMA((N_LANE,)),
            pltpu.SemaphoreType.DMA((N_LANE, N_HOP)),
        ],
        compiler_params=pltpu.CompilerParams(
            collective_id=0,
            vmem_limit_bytes=60 * 1024 * 1024,
        ),
    )(partial, resid, gamma2d)
